# baseline (speedup 1.0000x reference)
import math

import jax
import jax.numpy as jnp
from jax.experimental import pallas as pl
from jax.experimental.pallas import tpu as pltpu

_PATCH = 25
_SEG_PATCHES = 16
_N_PEAKS = 4
_LN10_INV = 1.0 / math.log(10.0)
_LN2_INV = 1.0 / math.log(2.0)
_EPS = 1e-5


def _shift_down(y):
    z = jnp.zeros_like(y[:1])
    return jnp.concatenate([z, y[:-1]], axis=0)


def _shift_up(y):
    z = jnp.zeros_like(y[:1])
    return jnp.concatenate([y[1:], z], axis=0)


def _seg_mean_matrix(t):
    r = jax.lax.broadcasted_iota(jnp.int32, (_N_PEAKS, t), 0)
    c = jax.lax.broadcasted_iota(jnp.int32, (_N_PEAKS, t), 1)
    msk = (c >= r * _SEG_PATCHES) & (c < (r + 1) * _SEG_PATCHES)
    return msk.astype(jnp.float32) * (1.0 / _SEG_PATCHES)


def _k1_body(x_ref, a_ref, w_ref, xreg_ref, xmax_ref, ya_ref, st_ref):
    v = x_ref[0]
    m = v[:, 0, :]
    s = v[:, 0, :]
    for k in range(1, _PATCH):
        vk = v[:, k, :]
        m = jnp.maximum(m, vk)
        s = s + vk
    xmax_ref[0] = m
    xreg_ref[0] = jnp.dot(_seg_mean_matrix(v.shape[0]), s,
                          preferred_element_type=jnp.float32) * (1.0 / _PATCH)

    ap = jnp.max(a_ref[0], axis=1, keepdims=True)
    ap = jnp.log(ap + 1.0) * _LN10_INV
    w = w_ref[...]
    y = (_shift_down(ap * w[0]) + ap * w[1] + _shift_up(ap * w[2]))
    ya_ref[0] = y
    st_ref[0] = jnp.concatenate(
        [jnp.sum(y, axis=0, keepdims=True),
         jnp.sum(y * y, axis=0, keepdims=True)], axis=0)


def _k2_body(xp_ref, ya_ref, bn_ref, wx_ref, wa_ref, j_ref, st_ref):
    bn = bn_ref[...]
    a = jnp.maximum(ya_ref[0] * bn[0:1, :] + bn[1:2, :], 0.0)
    xp = xp_ref[0]
    wx = wx_ref[...]
    wa = wa_ref[...]
    y0 = (jnp.dot(xp, wx[0], preferred_element_type=jnp.float32) +
          jnp.dot(a, wa[0], preferred_element_type=jnp.float32))
    y1 = (jnp.dot(xp, wx[1], preferred_element_type=jnp.float32) +
          jnp.dot(a, wa[1], preferred_element_type=jnp.float32))
    y2 = (jnp.dot(xp, wx[2], preferred_element_type=jnp.float32) +
          jnp.dot(a, wa[2], preferred_element_type=jnp.float32))
    j = _shift_down(y0) + y1 + _shift_up(y2)
    j_ref[0] = j
    st_ref[0] = jnp.concatenate(
        [jnp.sum(j, axis=0, keepdims=True),
         jnp.sum(j * j, axis=0, keepdims=True)], axis=0)


def _k3_body(j_ref, bn_ref, xreg_ref, o_ref):
    bn = bn_ref[...]
    jn = jnp.maximum(j_ref[0] * bn[0:1, :] + bn[1:2, :], 0.0)
    jr = jnp.dot(_seg_mean_matrix(jn.shape[0]), jn,
                 preferred_element_type=jnp.float32)
    jl = jnp.log(jr + 1.0) * _LN2_INV
    d = xreg_ref.shape[2]
    o_ref[0, :, :d] = xreg_ref[0]
    o_ref[0, :, d:] = jl


def _bn_affine(st, n):
    s = jnp.sum(st, axis=0)
    m = s[0] / n
    v = jnp.maximum(s[1] / n - m * m, 0.0)
    sc = jax.lax.rsqrt(v + _EPS)
    return jnp.stack([sc, -m * sc])


def kernel(x, atac, atac_conv_w3, joint_conv_w3_x, joint_conv_w3_a):
    B, L, D = x.shape
    T = L // _PATCH
    A = atac_conv_w3.shape[2]
    J = joint_conv_w3_x.shape[2]
    x4 = x.reshape(B, T, _PATCH, D)
    a3 = atac.reshape(B, T, _PATCH)

    xreg, xmax, ya, st1 = pl.pallas_call(
        _k1_body,
        out_shape=(jax.ShapeDtypeStruct((B, _N_PEAKS, D), jnp.float32),
                   jax.ShapeDtypeStruct((B, T, D), jnp.float32),
                   jax.ShapeDtypeStruct((B, T, A), jnp.float32),
                   jax.ShapeDtypeStruct((B, 2, A), jnp.float32)),
        grid=(B,),
        in_specs=[pl.BlockSpec((1, T, _PATCH, D), lambda b: (b, 0, 0, 0)),
                  pl.BlockSpec((1, T, _PATCH), lambda b: (b, 0, 0)),
                  pl.BlockSpec((3, 1, A), lambda b: (0, 0, 0))],
        out_specs=[pl.BlockSpec((1, _N_PEAKS, D), lambda b: (b, 0, 0)),
                   pl.BlockSpec((1, T, D), lambda b: (b, 0, 0)),
                   pl.BlockSpec((1, T, A), lambda b: (b, 0, 0)),
                   pl.BlockSpec((1, 2, A), lambda b: (b, 0, 0))],
        compiler_params=pltpu.CompilerParams(
            dimension_semantics=("parallel",),
            vmem_limit_bytes=64 * 1024 * 1024),
    )(x4, a3, atac_conv_w3)

    bn1 = _bn_affine(st1, B * T)

    j, st2 = pl.pallas_call(
        _k2_body,
        out_shape=(jax.ShapeDtypeStruct((B, T, J), jnp.float32),
                   jax.ShapeDtypeStruct((B, 2, J), jnp.float32)),
        grid=(B,),
        in_specs=[pl.BlockSpec((1, T, D), lambda b: (b, 0, 0)),
                  pl.BlockSpec((1, T, A), lambda b: (b, 0, 0)),
                  pl.BlockSpec((2, A), lambda b: (0, 0)),
                  pl.BlockSpec((3, D, J), lambda b: (0, 0, 0)),
                  pl.BlockSpec((3, A, J), lambda b: (0, 0, 0))],
        out_specs=[pl.BlockSpec((1, T, J), lambda b: (b, 0, 0)),
                   pl.BlockSpec((1, 2, J), lambda b: (b, 0, 0))],
        compiler_params=pltpu.CompilerParams(
            dimension_semantics=("parallel",)),
    )(xmax, ya, bn1, joint_conv_w3_x, joint_conv_w3_a)

    bn2 = _bn_affine(st2, B * T)

    out = pl.pallas_call(
        _k3_body,
        out_shape=jax.ShapeDtypeStruct((B, _N_PEAKS, D + J), jnp.float32),
        grid=(B,),
        in_specs=[pl.BlockSpec((1, T, J), lambda b: (b, 0, 0)),
                  pl.BlockSpec((2, J), lambda b: (0, 0)),
                  pl.BlockSpec((1, _N_PEAKS, D), lambda b: (b, 0, 0))],
        out_specs=pl.BlockSpec((1, _N_PEAKS, D + J), lambda b: (b, 0, 0)),
        compiler_params=pltpu.CompilerParams(
            dimension_semantics=("parallel",)),
    )(j, bn2, xreg)
    return out


# speedup vs baseline: 2.0403x; 2.0403x over previous
import math

import numpy as np

import jax
import jax.numpy as jnp
from jax.experimental import pallas as pl
from jax.experimental.pallas import tpu as pltpu

_PATCH = 25
_SEG = 400
_SEG_PATCHES = 16
_N_PEAKS = 4
_LN10_INV = 1.0 / math.log(10.0)
_LN2_INV = 1.0 / math.log(2.0)
_EPS = 1e-5


def _sh(y, s):
    return jnp.concatenate([y[s:], jnp.zeros((s, y.shape[1]), y.dtype)], axis=0)


def _seg_mean_matrix(t):
    r = jax.lax.broadcasted_iota(jnp.int32, (_N_PEAKS, t), 0)
    c = jax.lax.broadcasted_iota(jnp.int32, (_N_PEAKS, t), 1)
    msk = (c >= r * _SEG_PATCHES) & (c < (r + 1) * _SEG_PATCHES)
    return msk.astype(jnp.float32) * (1.0 / _SEG_PATCHES)


def _bn_affine(st_ref, n):
    s = jnp.sum(st_ref[...], axis=0) * (1.0 / n)
    mean = s[0:1, :]
    var = jnp.maximum(s[1:2, :] - mean * mean, 0.0)
    sc = jax.lax.rsqrt(var + _EPS)
    return sc, -mean * sc


def _k1_body(x_ref, a_ref, w_ref, sel_ref, seg_ref,
             xreg_ref, xmax_ref, ya_ref, st_ref):
    v = x_ref[0]
    m = jnp.maximum(v, _sh(v, 1))
    m = jnp.maximum(m, _sh(m, 2))
    m = jnp.maximum(m, _sh(m, 4))
    m = jnp.maximum(m, _sh(m, 8))
    m = jnp.maximum(m, _sh(m, 9))
    xmax_ref[0] = jnp.dot(sel_ref[...], m, preferred_element_type=jnp.float32)
    xreg_ref[0] = jnp.dot(seg_ref[...], v, preferred_element_type=jnp.float32)

    ap = jnp.max(a_ref[0], axis=1, keepdims=True)
    ap = jnp.log(ap + 1.0) * _LN10_INV
    w = w_ref[...]
    y0 = ap * w[0]
    y1 = ap * w[1]
    y2 = ap * w[2]
    z = jnp.zeros_like(y1[:1])
    y = (jnp.concatenate([z, y0[:-1]], axis=0) + y1 +
         jnp.concatenate([y2[1:], z], axis=0))
    ya_ref[0] = y
    st_ref[0] = jnp.concatenate(
        [jnp.sum(y, axis=0, keepdims=True),
         jnp.sum(y * y, axis=0, keepdims=True)], axis=0)


def _k2_body(xp_ref, ya_ref, st1_ref, wx_ref, wa_ref, j_ref, st_ref):
    n = st1_ref.shape[0] * ya_ref.shape[1]
    sc, bias = _bn_affine(st1_ref, n)
    a = jnp.maximum(ya_ref[0] * sc + bias, 0.0)
    xp = xp_ref[0]
    wx = wx_ref[...]
    wa = wa_ref[...]
    y0 = (jnp.dot(xp, wx[0], preferred_element_type=jnp.float32) +
          jnp.dot(a, wa[0], preferred_element_type=jnp.float32))
    y1 = (jnp.dot(xp, wx[1], preferred_element_type=jnp.float32) +
          jnp.dot(a, wa[1], preferred_element_type=jnp.float32))
    y2 = (jnp.dot(xp, wx[2], preferred_element_type=jnp.float32) +
          jnp.dot(a, wa[2], preferred_element_type=jnp.float32))
    z = jnp.zeros_like(y1[:1])
    j = (jnp.concatenate([z, y0[:-1]], axis=0) + y1 +
         jnp.concatenate([y2[1:], z], axis=0))
    j_ref[0] = j
    st_ref[0] = jnp.concatenate(
        [jnp.sum(j, axis=0, keepdims=True),
         jnp.sum(j * j, axis=0, keepdims=True)], axis=0)


def _k3_body(j_ref, st2_ref, xreg_ref, o_ref):
    n = st2_ref.shape[0] * j_ref.shape[1]
    sc, bias = _bn_affine(st2_ref, n)
    jn = jnp.maximum(j_ref[0] * sc + bias, 0.0)
    jr = jnp.dot(_seg_mean_matrix(jn.shape[0]), jn,
                 preferred_element_type=jnp.float32)
    jl = jnp.log(jr + 1.0) * _LN2_INV
    d = xreg_ref.shape[2]
    o_ref[0, :, :d] = xreg_ref[0]
    o_ref[0, :, d:] = jl


def kernel(x, atac, atac_conv_w3, joint_conv_w3_x, joint_conv_w3_a):
    B, L, D = x.shape
    T = L // _PATCH
    A = atac_conv_w3.shape[2]
    J = joint_conv_w3_x.shape[2]
    a3 = atac.reshape(B, T, _PATCH)

    tt = np.arange(L)
    sel = (tt[None, :] == (np.arange(T) * _PATCH)[:, None]).astype(np.float32)
    seg = ((tt[None, :] // _SEG) == np.arange(_N_PEAKS)[:, None]).astype(
        np.float32) * (1.0 / _SEG)
    sel = jnp.asarray(sel)
    seg = jnp.asarray(seg)

    xreg, xmax, ya, st1 = pl.pallas_call(
        _k1_body,
        out_shape=(jax.ShapeDtypeStruct((B, _N_PEAKS, D), jnp.float32),
                   jax.ShapeDtypeStruct((B, T, D), jnp.float32),
                   jax.ShapeDtypeStruct((B, T, A), jnp.float32),
                   jax.ShapeDtypeStruct((B, 2, A), jnp.float32)),
        grid=(B,),
        in_specs=[pl.BlockSpec((1, L, D), lambda b: (b, 0, 0)),
                  pl.BlockSpec((1, T, _PATCH), lambda b: (b, 0, 0)),
                  pl.BlockSpec((3, 1, A), lambda b: (0, 0, 0)),
                  pl.BlockSpec((T, L), lambda b: (0, 0)),
                  pl.BlockSpec((_N_PEAKS, L), lambda b: (0, 0))],
        out_specs=[pl.BlockSpec((1, _N_PEAKS, D), lambda b: (b, 0, 0)),
                   pl.BlockSpec((1, T, D), lambda b: (b, 0, 0)),
                   pl.BlockSpec((1, T, A), lambda b: (b, 0, 0)),
                   pl.BlockSpec((1, 2, A), lambda b: (b, 0, 0))],
        compiler_params=pltpu.CompilerParams(
            dimension_semantics=("parallel",),
            vmem_limit_bytes=64 * 1024 * 1024),
    )(x, a3, atac_conv_w3, sel, seg)

    j, st2 = pl.pallas_call(
        _k2_body,
        out_shape=(jax.ShapeDtypeStruct((B, T, J), jnp.float32),
                   jax.ShapeDtypeStruct((B, 2, J), jnp.float32)),
        grid=(B,),
        in_specs=[pl.BlockSpec((1, T, D), lambda b: (b, 0, 0)),
                  pl.BlockSpec((1, T, A), lambda b: (b, 0, 0)),
                  pl.BlockSpec((B, 2, A), lambda b: (0, 0, 0)),
                  pl.BlockSpec((3, D, J), lambda b: (0, 0, 0)),
                  pl.BlockSpec((3, A, J), lambda b: (0, 0, 0))],
        out_specs=[pl.BlockSpec((1, T, J), lambda b: (b, 0, 0)),
                   pl.BlockSpec((1, 2, J), lambda b: (b, 0, 0))],
        compiler_params=pltpu.CompilerParams(
            dimension_semantics=("parallel",)),
    )(xmax, ya, st1, joint_conv_w3_x, joint_conv_w3_a)

    out = pl.pallas_call(
        _k3_body,
        out_shape=jax.ShapeDtypeStruct((B, _N_PEAKS, D + J), jnp.float32),
        grid=(B,),
        in_specs=[pl.BlockSpec((1, T, J), lambda b: (b, 0, 0)),
                  pl.BlockSpec((B, 2, J), lambda b: (0, 0, 0)),
                  pl.BlockSpec((1, _N_PEAKS, D), lambda b: (b, 0, 0))],
        out_specs=pl.BlockSpec((1, _N_PEAKS, D + J), lambda b: (b, 0, 0)),
        compiler_params=pltpu.CompilerParams(
            dimension_semantics=("parallel",)),
    )(j, st2, xreg)
    return out


# speedup vs baseline: 2.5884x; 1.2686x over previous
import math

import numpy as np

import jax
import jax.numpy as jnp
from jax.experimental import pallas as pl
from jax.experimental.pallas import tpu as pltpu

_PATCH = 25
_SEG = 400
_SEG_PATCHES = 16
_N_PEAKS = 4
_LN10_INV = 1.0 / math.log(10.0)
_LN2_INV = 1.0 / math.log(2.0)
_EPS = 1e-5


def _sh(y, s):
    return jnp.concatenate([y[s:], jnp.zeros((s, y.shape[1]), y.dtype)], axis=0)


def _shift_down(y):
    z = jnp.zeros_like(y[:1])
    return jnp.concatenate([z, y[:-1]], axis=0)


def _shift_up(y):
    z = jnp.zeros_like(y[:1])
    return jnp.concatenate([y[1:], z], axis=0)


def _seg_mean_matrix(t):
    r = jax.lax.broadcasted_iota(jnp.int32, (_N_PEAKS, t), 0)
    c = jax.lax.broadcasted_iota(jnp.int32, (_N_PEAKS, t), 1)
    msk = (c >= r * _SEG_PATCHES) & (c < (r + 1) * _SEG_PATCHES)
    return msk.astype(jnp.float32) * (1.0 / _SEG_PATCHES)


def _k0_body(a_ref, w_ref, bn_ref):
    af = jnp.max(a_ref[...], axis=2)
    af = jnp.log(af + 1.0) * _LN10_INV
    z = jnp.zeros_like(af[:, :1])
    u0 = jnp.concatenate([z, af[:, :-1]], axis=1)
    u2 = jnp.concatenate([af[:, 1:], z], axis=1)
    n = af.shape[0] * af.shape[1]
    inv_n = 1.0 / n
    s0 = jnp.sum(u0) * inv_n
    s1 = jnp.sum(af) * inv_n
    s2 = jnp.sum(u2) * inv_n
    q00 = jnp.sum(u0 * u0) * inv_n
    q11 = jnp.sum(af * af) * inv_n
    q22 = jnp.sum(u2 * u2) * inv_n
    q01 = jnp.sum(u0 * af) * inv_n
    q02 = jnp.sum(u0 * u2) * inv_n
    q12 = jnp.sum(af * u2) * inv_n
    w = w_ref[...]
    w0, w1, w2 = w[0], w[1], w[2]
    mean = w0 * s0 + w1 * s1 + w2 * s2
    ey2 = (w0 * w0 * q00 + w1 * w1 * q11 + w2 * w2 * q22 +
           2.0 * (w0 * w1 * q01 + w0 * w2 * q02 + w1 * w2 * q12))
    var = jnp.maximum(ey2 - mean * mean, 0.0)
    sc = jax.lax.rsqrt(var + _EPS)
    bn_ref[...] = jnp.concatenate([sc, -mean * sc], axis=0)


def _k1_body(x_ref, a_ref, w_ref, bn_ref, sel_ref, seg_ref,
             wx_ref, wa_ref, xreg_ref, j_ref, st_ref):
    v = x_ref[0]
    m = jnp.maximum(v, _sh(v, 1))
    m = jnp.maximum(m, _sh(m, 2))
    m = jnp.maximum(m, _sh(m, 4))
    m = jnp.maximum(m, _sh(m, 8))
    m = jnp.maximum(m, _sh(m, 9))
    xmax = jnp.dot(sel_ref[...], m, preferred_element_type=jnp.float32)
    xreg_ref[0] = jnp.dot(seg_ref[...], v, preferred_element_type=jnp.float32)

    ap = jnp.max(a_ref[0], axis=1, keepdims=True)
    ap = jnp.log(ap + 1.0) * _LN10_INV
    w = w_ref[...]
    y = (_shift_down(ap * w[0]) + ap * w[1] + _shift_up(ap * w[2]))
    bn = bn_ref[...]
    a = jnp.maximum(y * bn[0:1, :] + bn[1:2, :], 0.0)

    wx = wx_ref[...]
    wa = wa_ref[...]
    y0 = (jnp.dot(xmax, wx[0], preferred_element_type=jnp.float32) +
          jnp.dot(a, wa[0], preferred_element_type=jnp.float32))
    y1 = (jnp.dot(xmax, wx[1], preferred_element_type=jnp.float32) +
          jnp.dot(a, wa[1], preferred_element_type=jnp.float32))
    y2 = (jnp.dot(xmax, wx[2], preferred_element_type=jnp.float32) +
          jnp.dot(a, wa[2], preferred_element_type=jnp.float32))
    j = _shift_down(y0) + y1 + _shift_up(y2)
    j_ref[0] = j
    st_ref[0] = jnp.concatenate(
        [jnp.sum(j, axis=0, keepdims=True),
         jnp.sum(j * j, axis=0, keepdims=True)], axis=0)


def _k2_body(j_ref, st_ref, xreg_ref, o_ref):
    n = st_ref.shape[0] * j_ref.shape[1]
    s = jnp.sum(st_ref[...], axis=0) * (1.0 / n)
    mean = s[0:1, :]
    var = jnp.maximum(s[1:2, :] - mean * mean, 0.0)
    sc = jax.lax.rsqrt(var + _EPS)
    jn = jnp.maximum(j_ref[0] * sc - mean * sc, 0.0)
    jr = jnp.dot(_seg_mean_matrix(jn.shape[0]), jn,
                 preferred_element_type=jnp.float32)
    jl = jnp.log(jr + 1.0) * _LN2_INV
    d = xreg_ref.shape[2]
    o_ref[0, :, :d] = xreg_ref[0]
    o_ref[0, :, d:] = jl


def kernel(x, atac, atac_conv_w3, joint_conv_w3_x, joint_conv_w3_a):
    B, L, D = x.shape
    T = L // _PATCH
    A = atac_conv_w3.shape[2]
    J = joint_conv_w3_x.shape[2]
    a3 = atac.reshape(B, T, _PATCH)

    tt = np.arange(L)
    sel = (tt[None, :] == (np.arange(T) * _PATCH)[:, None]).astype(np.float32)
    seg = ((tt[None, :] // _SEG) == np.arange(_N_PEAKS)[:, None]).astype(
        np.float32) * (1.0 / _SEG)
    sel = jnp.asarray(sel)
    seg = jnp.asarray(seg)

    bn1 = pl.pallas_call(
        _k0_body,
        out_shape=jax.ShapeDtypeStruct((2, A), jnp.float32),
        grid=(1,),
        in_specs=[pl.BlockSpec((B, T, _PATCH), lambda i: (0, 0, 0)),
                  pl.BlockSpec((3, 1, A), lambda i: (0, 0, 0))],
        out_specs=pl.BlockSpec((2, A), lambda i: (0, 0)),
    )(a3, atac_conv_w3)

    xreg, j, st2 = pl.pallas_call(
        _k1_body,
        out_shape=(jax.ShapeDtypeStruct((B, _N_PEAKS, D), jnp.float32),
                   jax.ShapeDtypeStruct((B, T, J), jnp.float32),
                   jax.ShapeDtypeStruct((B, 2, J), jnp.float32)),
        grid=(B,),
        in_specs=[pl.BlockSpec((1, L, D), lambda b: (b, 0, 0)),
                  pl.BlockSpec((1, T, _PATCH), lambda b: (b, 0, 0)),
                  pl.BlockSpec((3, 1, A), lambda b: (0, 0, 0)),
                  pl.BlockSpec((2, A), lambda b: (0, 0)),
                  pl.BlockSpec((T, L), lambda b: (0, 0)),
                  pl.BlockSpec((_N_PEAKS, L), lambda b: (0, 0)),
                  pl.BlockSpec((3, D, J), lambda b: (0, 0, 0)),
                  pl.BlockSpec((3, A, J), lambda b: (0, 0, 0))],
        out_specs=[pl.BlockSpec((1, _N_PEAKS, D), lambda b: (b, 0, 0)),
                   pl.BlockSpec((1, T, J), lambda b: (b, 0, 0)),
                   pl.BlockSpec((1, 2, J), lambda b: (b, 0, 0))],
        compiler_params=pltpu.CompilerParams(
            dimension_semantics=("parallel",),
            vmem_limit_bytes=64 * 1024 * 1024),
    )(x, a3, atac_conv_w3, bn1, sel, seg, joint_conv_w3_x, joint_conv_w3_a)

    out = pl.pallas_call(
        _k2_body,
        out_shape=jax.ShapeDtypeStruct((B, _N_PEAKS, D + J), jnp.float32),
        grid=(B,),
        in_specs=[pl.BlockSpec((1, T, J), lambda b: (b, 0, 0)),
                  pl.BlockSpec((B, 2, J), lambda b: (0, 0, 0)),
                  pl.BlockSpec((1, _N_PEAKS, D), lambda b: (b, 0, 0))],
        out_specs=pl.BlockSpec((1, _N_PEAKS, D + J), lambda b: (b, 0, 0)),
        compiler_params=pltpu.CompilerParams(
            dimension_semantics=("parallel",)),
    )(j, st2, xreg)
    return out


# speedup vs baseline: 4.6567x; 1.7991x over previous
import math

import numpy as np

import jax
import jax.numpy as jnp
from jax.experimental import pallas as pl
from jax.experimental.pallas import tpu as pltpu

_PATCH = 25
_SEG = 400
_SEG_PATCHES = 16
_N_PEAKS = 4
_LN10_INV = 1.0 / math.log(10.0)
_LN2_INV = 1.0 / math.log(2.0)
_EPS = 1e-5


def _sh(y, s):
    return jnp.concatenate([y[s:], jnp.zeros((s, y.shape[1]), y.dtype)], axis=0)


def _shift_down(y):
    z = jnp.zeros_like(y[:1])
    return jnp.concatenate([z, y[:-1]], axis=0)


def _shift_up(y):
    z = jnp.zeros_like(y[:1])
    return jnp.concatenate([y[1:], z], axis=0)


def _seg_mean_matrix(t):
    r = jax.lax.broadcasted_iota(jnp.int32, (_N_PEAKS, t), 0)
    c = jax.lax.broadcasted_iota(jnp.int32, (_N_PEAKS, t), 1)
    msk = (c >= r * _SEG_PATCHES) & (c < (r + 1) * _SEG_PATCHES)
    return msk.astype(jnp.float32) * (1.0 / _SEG_PATCHES)


def _k0_body(a_ref, w_ref, bn_ref):
    af = jnp.max(a_ref[...], axis=2)
    af = jnp.log(af + 1.0) * _LN10_INV
    z = jnp.zeros_like(af[:, :1])
    u0 = jnp.concatenate([z, af[:, :-1]], axis=1)
    u2 = jnp.concatenate([af[:, 1:], z], axis=1)
    n = af.shape[0] * af.shape[1]
    inv_n = 1.0 / n
    s0 = jnp.sum(u0) * inv_n
    s1 = jnp.sum(af) * inv_n
    s2 = jnp.sum(u2) * inv_n
    q00 = jnp.sum(u0 * u0) * inv_n
    q11 = jnp.sum(af * af) * inv_n
    q22 = jnp.sum(u2 * u2) * inv_n
    q01 = jnp.sum(u0 * af) * inv_n
    q02 = jnp.sum(u0 * u2) * inv_n
    q12 = jnp.sum(af * u2) * inv_n
    w = w_ref[...]
    w0, w1, w2 = w[0], w[1], w[2]
    mean = w0 * s0 + w1 * s1 + w2 * s2
    ey2 = (w0 * w0 * q00 + w1 * w1 * q11 + w2 * w2 * q22 +
           2.0 * (w0 * w1 * q01 + w0 * w2 * q02 + w1 * w2 * q12))
    var = jnp.maximum(ey2 - mean * mean, 0.0)
    sc = jax.lax.rsqrt(var + _EPS)
    bn_ref[...] = jnp.concatenate([sc, -mean * sc], axis=0)


def _k1_body(x_ref, a_ref, w_ref, bn_ref, sel_ref, seg_ref,
             wx_ref, wa_ref, xreg_ref, j_ref, st_ref):
    w = w_ref[...]
    bn = bn_ref[...]
    wx = wx_ref[...]
    wa = wa_ref[...]
    ssum = None
    ssq = None
    for g in range(x_ref.shape[0]):
        v = x_ref[g]
        m = jnp.maximum(v, _sh(v, 1))
        m = jnp.maximum(m, _sh(m, 2))
        m = jnp.maximum(m, _sh(m, 4))
        m = jnp.maximum(jnp.maximum(m, _sh(m, 8)),
                        jnp.maximum(_sh(m, 16), _sh(v, 24)))
        xmax = jnp.dot(sel_ref[...], m, preferred_element_type=jnp.float32)
        xreg_ref[g] = jnp.dot(seg_ref[...], v,
                              preferred_element_type=jnp.float32)

        ap = jnp.max(a_ref[g], axis=1, keepdims=True)
        ap = jnp.log(ap + 1.0) * _LN10_INV
        y = (_shift_down(ap * w[0]) + ap * w[1] + _shift_up(ap * w[2]))
        a = jnp.maximum(y * bn[0:1, :] + bn[1:2, :], 0.0)

        y0 = (jnp.dot(xmax, wx[0], preferred_element_type=jnp.float32) +
              jnp.dot(a, wa[0], preferred_element_type=jnp.float32))
        y1 = (jnp.dot(xmax, wx[1], preferred_element_type=jnp.float32) +
              jnp.dot(a, wa[1], preferred_element_type=jnp.float32))
        y2 = (jnp.dot(xmax, wx[2], preferred_element_type=jnp.float32) +
              jnp.dot(a, wa[2], preferred_element_type=jnp.float32))
        j = _shift_down(y0) + y1 + _shift_up(y2)
        j_ref[g] = j
        js = jnp.sum(j, axis=0, keepdims=True)
        jq = jnp.sum(j * j, axis=0, keepdims=True)
        ssum = js if ssum is None else ssum + js
        ssq = jq if ssq is None else ssq + jq
    st_ref[0] = jnp.concatenate([ssum, ssq], axis=0)


def _k2_body(j_ref, st_ref, xreg_ref, o_ref):
    g_blk, t, _ = j_ref.shape
    n = st_ref.shape[0] * g_blk * t
    s = jnp.sum(st_ref[...], axis=0) * (1.0 / n)
    mean = s[0:1, :]
    var = jnp.maximum(s[1:2, :] - mean * mean, 0.0)
    sc = jax.lax.rsqrt(var + _EPS)
    bias = -mean * sc
    d = xreg_ref.shape[2]
    msk = _seg_mean_matrix(t)
    for g in range(g_blk):
        jn = jnp.maximum(j_ref[g] * sc + bias, 0.0)
        jr = jnp.dot(msk, jn, preferred_element_type=jnp.float32)
        jl = jnp.log(jr + 1.0) * _LN2_INV
        o_ref[g, :, :d] = xreg_ref[g]
        o_ref[g, :, d:] = jl


def kernel(x, atac, atac_conv_w3, joint_conv_w3_x, joint_conv_w3_a):
    B, L, D = x.shape
    T = L // _PATCH
    A = atac_conv_w3.shape[2]
    J = joint_conv_w3_x.shape[2]
    a3 = atac.reshape(B, T, _PATCH)

    tt = np.arange(L)
    sel = (tt[None, :] == (np.arange(T) * _PATCH)[:, None]).astype(np.float32)
    seg = ((tt[None, :] // _SEG) == np.arange(_N_PEAKS)[:, None]).astype(
        np.float32) * (1.0 / _SEG)
    sel = jnp.asarray(sel)
    seg = jnp.asarray(seg)

    bn1 = pl.pallas_call(
        _k0_body,
        out_shape=jax.ShapeDtypeStruct((2, A), jnp.float32),
        grid=(1,),
        in_specs=[pl.BlockSpec((B, T, _PATCH), lambda i: (0, 0, 0)),
                  pl.BlockSpec((3, 1, A), lambda i: (0, 0, 0))],
        out_specs=pl.BlockSpec((2, A), lambda i: (0, 0)),
    )(a3, atac_conv_w3)

    G = 4
    NG = B // G
    xreg, j, st2 = pl.pallas_call(
        _k1_body,
        out_shape=(jax.ShapeDtypeStruct((B, _N_PEAKS, D), jnp.float32),
                   jax.ShapeDtypeStruct((B, T, J), jnp.float32),
                   jax.ShapeDtypeStruct((NG, 2, J), jnp.float32)),
        grid=(NG,),
        in_specs=[pl.BlockSpec((G, L, D), lambda b: (b, 0, 0)),
                  pl.BlockSpec((G, T, _PATCH), lambda b: (b, 0, 0)),
                  pl.BlockSpec((3, 1, A), lambda b: (0, 0, 0)),
                  pl.BlockSpec((2, A), lambda b: (0, 0)),
                  pl.BlockSpec((T, L), lambda b: (0, 0)),
                  pl.BlockSpec((_N_PEAKS, L), lambda b: (0, 0)),
                  pl.BlockSpec((3, D, J), lambda b: (0, 0, 0)),
                  pl.BlockSpec((3, A, J), lambda b: (0, 0, 0))],
        out_specs=[pl.BlockSpec((G, _N_PEAKS, D), lambda b: (b, 0, 0)),
                   pl.BlockSpec((G, T, J), lambda b: (b, 0, 0)),
                   pl.BlockSpec((1, 2, J), lambda b: (b, 0, 0))],
        compiler_params=pltpu.CompilerParams(
            dimension_semantics=("parallel",),
            vmem_limit_bytes=100 * 1024 * 1024),
    )(x, a3, atac_conv_w3, bn1, sel, seg, joint_conv_w3_x, joint_conv_w3_a)

    out = pl.pallas_call(
        _k2_body,
        out_shape=jax.ShapeDtypeStruct((B, _N_PEAKS, D + J), jnp.float32),
        grid=(NG,),
        in_specs=[pl.BlockSpec((G, T, J), lambda b: (b, 0, 0)),
                  pl.BlockSpec((NG, 2, J), lambda b: (0, 0, 0)),
                  pl.BlockSpec((G, _N_PEAKS, D), lambda b: (b, 0, 0))],
        out_specs=pl.BlockSpec((G, _N_PEAKS, D + J), lambda b: (b, 0, 0)),
        compiler_params=pltpu.CompilerParams(
            dimension_semantics=("parallel",)),
    )(j, st2, xreg)
    return out


# speedup vs baseline: 4.9845x; 1.0704x over previous
import math

import numpy as np

import jax
import jax.numpy as jnp
from jax.experimental import pallas as pl
from jax.experimental.pallas import tpu as pltpu

_PATCH = 25
_SEG = 400
_SEG_PATCHES = 16
_N_PEAKS = 4
_LN10_INV = 1.0 / math.log(10.0)
_LN2_INV = 1.0 / math.log(2.0)
_EPS = 1e-5


def _sh(y, s):
    return jnp.concatenate([y[s:], jnp.zeros((s, y.shape[1]), y.dtype)], axis=0)


def _shift_down(y):
    z = jnp.zeros_like(y[:1])
    return jnp.concatenate([z, y[:-1]], axis=0)


def _shift_up(y):
    z = jnp.zeros_like(y[:1])
    return jnp.concatenate([y[1:], z], axis=0)


def _seg_mean_matrix(t):
    r = jax.lax.broadcasted_iota(jnp.int32, (_N_PEAKS, t), 0)
    c = jax.lax.broadcasted_iota(jnp.int32, (_N_PEAKS, t), 1)
    msk = (c >= r * _SEG_PATCHES) & (c < (r + 1) * _SEG_PATCHES)
    return msk.astype(jnp.float32) * (1.0 / _SEG_PATCHES)


def _k0_body(a_ref, w_ref, bn_ref):
    af = jnp.max(a_ref[...], axis=2)
    af = jnp.log(af + 1.0) * _LN10_INV
    z = jnp.zeros_like(af[:, :1])
    u0 = jnp.concatenate([z, af[:, :-1]], axis=1)
    u2 = jnp.concatenate([af[:, 1:], z], axis=1)
    n = af.shape[0] * af.shape[1]
    inv_n = 1.0 / n
    s0 = jnp.sum(u0) * inv_n
    s1 = jnp.sum(af) * inv_n
    s2 = jnp.sum(u2) * inv_n
    q00 = jnp.sum(u0 * u0) * inv_n
    q11 = jnp.sum(af * af) * inv_n
    q22 = jnp.sum(u2 * u2) * inv_n
    q01 = jnp.sum(u0 * af) * inv_n
    q02 = jnp.sum(u0 * u2) * inv_n
    q12 = jnp.sum(af * u2) * inv_n
    w = w_ref[...]
    w0, w1, w2 = w[0], w[1], w[2]
    mean = w0 * s0 + w1 * s1 + w2 * s2
    ey2 = (w0 * w0 * q00 + w1 * w1 * q11 + w2 * w2 * q22 +
           2.0 * (w0 * w1 * q01 + w0 * w2 * q02 + w1 * w2 * q12))
    var = jnp.maximum(ey2 - mean * mean, 0.0)
    sc = jax.lax.rsqrt(var + _EPS)
    bn_ref[...] = jnp.concatenate([sc, -mean * sc], axis=0)


def _k1_body(x_ref, a_ref, w_ref, bn_ref, sel_ref, seg_ref,
             wx_ref, wa_ref, xreg_ref, j_ref, st_ref):
    w = w_ref[...]
    bn = bn_ref[...]
    wx = wx_ref[...]
    wa = wa_ref[...]
    ssum = None
    ssq = None
    for g in range(x_ref.shape[0]):
        v = x_ref[g]
        m = jnp.maximum(v, _sh(v, 1))
        m = jnp.maximum(m, _sh(m, 2))
        m = jnp.maximum(m, _sh(m, 4))
        m = jnp.maximum(jnp.maximum(m, _sh(m, 8)),
                        jnp.maximum(_sh(m, 16), _sh(v, 24)))
        xmax = jnp.dot(sel_ref[...], m, preferred_element_type=jnp.float32)
        xreg_ref[g] = jnp.dot(seg_ref[...], v,
                              preferred_element_type=jnp.float32)

        ap = jnp.max(a_ref[g], axis=1, keepdims=True)
        ap = jnp.log(ap + 1.0) * _LN10_INV
        y = (_shift_down(ap * w[0]) + ap * w[1] + _shift_up(ap * w[2]))
        a = jnp.maximum(y * bn[0:1, :] + bn[1:2, :], 0.0)

        y0 = (jnp.dot(xmax, wx[0], preferred_element_type=jnp.float32) +
              jnp.dot(a, wa[0], preferred_element_type=jnp.float32))
        y1 = (jnp.dot(xmax, wx[1], preferred_element_type=jnp.float32) +
              jnp.dot(a, wa[1], preferred_element_type=jnp.float32))
        y2 = (jnp.dot(xmax, wx[2], preferred_element_type=jnp.float32) +
              jnp.dot(a, wa[2], preferred_element_type=jnp.float32))
        j = _shift_down(y0) + y1 + _shift_up(y2)
        j_ref[g] = j
        js = jnp.sum(j, axis=0, keepdims=True)
        jq = jnp.sum(j * j, axis=0, keepdims=True)
        ssum = js if ssum is None else ssum + js
        ssq = jq if ssq is None else ssq + jq
    st_ref[0] = jnp.concatenate([ssum, ssq], axis=0)


def _k2_body(j_ref, st_ref, xreg_ref, o_ref):
    g_blk, t, _ = j_ref.shape
    n = st_ref.shape[0] * g_blk * t
    s = jnp.sum(st_ref[...], axis=0) * (1.0 / n)
    mean = s[0:1, :]
    var = jnp.maximum(s[1:2, :] - mean * mean, 0.0)
    sc = jax.lax.rsqrt(var + _EPS)
    bias = -mean * sc
    d = xreg_ref.shape[2]
    msk = _seg_mean_matrix(t)
    for g in range(g_blk):
        jn = jnp.maximum(j_ref[g] * sc + bias, 0.0)
        jr = jnp.dot(msk, jn, preferred_element_type=jnp.float32)
        jl = jnp.log(jr + 1.0) * _LN2_INV
        o_ref[g, :, :d] = xreg_ref[g]
        o_ref[g, :, d:] = jl


def kernel(x, atac, atac_conv_w3, joint_conv_w3_x, joint_conv_w3_a):
    B, L, D = x.shape
    T = L // _PATCH
    A = atac_conv_w3.shape[2]
    J = joint_conv_w3_x.shape[2]
    a3 = atac.reshape(B, T, _PATCH)

    tt = np.arange(L)
    sel = (tt[None, :] == (np.arange(T) * _PATCH)[:, None]).astype(np.float32)
    seg = ((tt[None, :] // _SEG) == np.arange(_N_PEAKS)[:, None]).astype(
        np.float32) * (1.0 / _SEG)
    sel = jnp.asarray(sel)
    seg = jnp.asarray(seg)

    bn1 = pl.pallas_call(
        _k0_body,
        out_shape=jax.ShapeDtypeStruct((2, A), jnp.float32),
        grid=(1,),
        in_specs=[pl.BlockSpec((B, T, _PATCH), lambda i: (0, 0, 0)),
                  pl.BlockSpec((3, 1, A), lambda i: (0, 0, 0))],
        out_specs=pl.BlockSpec((2, A), lambda i: (0, 0)),
    )(a3, atac_conv_w3)

    G = next(g for g in (8, 4, 2, 1) if B % g == 0)
    NG = B // G
    xreg, j, st2 = pl.pallas_call(
        _k1_body,
        out_shape=(jax.ShapeDtypeStruct((B, _N_PEAKS, D), jnp.float32),
                   jax.ShapeDtypeStruct((B, T, J), jnp.float32),
                   jax.ShapeDtypeStruct((NG, 2, J), jnp.float32)),
        grid=(NG,),
        in_specs=[pl.BlockSpec((G, L, D), lambda b: (b, 0, 0)),
                  pl.BlockSpec((G, T, _PATCH), lambda b: (b, 0, 0)),
                  pl.BlockSpec((3, 1, A), lambda b: (0, 0, 0)),
                  pl.BlockSpec((2, A), lambda b: (0, 0)),
                  pl.BlockSpec((T, L), lambda b: (0, 0)),
                  pl.BlockSpec((_N_PEAKS, L), lambda b: (0, 0)),
                  pl.BlockSpec((3, D, J), lambda b: (0, 0, 0)),
                  pl.BlockSpec((3, A, J), lambda b: (0, 0, 0))],
        out_specs=[pl.BlockSpec((G, _N_PEAKS, D), lambda b: (b, 0, 0)),
                   pl.BlockSpec((G, T, J), lambda b: (b, 0, 0)),
                   pl.BlockSpec((1, 2, J), lambda b: (b, 0, 0))],
        compiler_params=pltpu.CompilerParams(
            dimension_semantics=("parallel",),
            vmem_limit_bytes=100 * 1024 * 1024),
    )(x, a3, atac_conv_w3, bn1, sel, seg, joint_conv_w3_x, joint_conv_w3_a)

    out = pl.pallas_call(
        _k2_body,
        out_shape=jax.ShapeDtypeStruct((B, _N_PEAKS, D + J), jnp.float32),
        grid=(NG,),
        in_specs=[pl.BlockSpec((G, T, J), lambda b: (b, 0, 0)),
                  pl.BlockSpec((NG, 2, J), lambda b: (0, 0, 0)),
                  pl.BlockSpec((G, _N_PEAKS, D), lambda b: (b, 0, 0))],
        out_specs=pl.BlockSpec((G, _N_PEAKS, D + J), lambda b: (b, 0, 0)),
        compiler_params=pltpu.CompilerParams(
            dimension_semantics=("parallel",)),
    )(j, st2, xreg)
    return out


# speedup vs baseline: 5.3605x; 1.0754x over previous
import math

import numpy as np

import jax
import jax.numpy as jnp
from jax.experimental import pallas as pl
from jax.experimental.pallas import tpu as pltpu

_PATCH = 25
_SEG = 400
_SEG_PATCHES = 16
_N_PEAKS = 4
_LN10_INV = 1.0 / math.log(10.0)
_LN2_INV = 1.0 / math.log(2.0)
_EPS = 1e-5


def _sh(y, s):
    return jnp.concatenate([y[s:], jnp.zeros((s, y.shape[1]), y.dtype)], axis=0)


def _shift_down(y):
    z = jnp.zeros_like(y[:1])
    return jnp.concatenate([z, y[:-1]], axis=0)


def _shift_up(y):
    z = jnp.zeros_like(y[:1])
    return jnp.concatenate([y[1:], z], axis=0)


def _seg_mean_matrix(t):
    r = jax.lax.broadcasted_iota(jnp.int32, (_N_PEAKS, t), 0)
    c = jax.lax.broadcasted_iota(jnp.int32, (_N_PEAKS, t), 1)
    msk = (c >= r * _SEG_PATCHES) & (c < (r + 1) * _SEG_PATCHES)
    return msk.astype(jnp.float32) * (1.0 / _SEG_PATCHES)


def _k0_body(a_ref, w_ref, bn_ref):
    af = jnp.max(a_ref[...], axis=2)
    af = jnp.log(af + 1.0) * _LN10_INV
    z = jnp.zeros_like(af[:, :1])
    u0 = jnp.concatenate([z, af[:, :-1]], axis=1)
    u2 = jnp.concatenate([af[:, 1:], z], axis=1)
    n = af.shape[0] * af.shape[1]
    inv_n = 1.0 / n
    s0 = jnp.sum(u0) * inv_n
    s1 = jnp.sum(af) * inv_n
    s2 = jnp.sum(u2) * inv_n
    q00 = jnp.sum(u0 * u0) * inv_n
    q11 = jnp.sum(af * af) * inv_n
    q22 = jnp.sum(u2 * u2) * inv_n
    q01 = jnp.sum(u0 * af) * inv_n
    q02 = jnp.sum(u0 * u2) * inv_n
    q12 = jnp.sum(af * u2) * inv_n
    w = w_ref[...]
    w0, w1, w2 = w[0], w[1], w[2]
    mean = w0 * s0 + w1 * s1 + w2 * s2
    ey2 = (w0 * w0 * q00 + w1 * w1 * q11 + w2 * w2 * q22 +
           2.0 * (w0 * w1 * q01 + w0 * w2 * q02 + w1 * w2 * q12))
    var = jnp.maximum(ey2 - mean * mean, 0.0)
    sc = jax.lax.rsqrt(var + _EPS)
    bn_ref[...] = jnp.concatenate([sc, -mean * sc], axis=0)


def _k1_body(x_ref, a_ref, w_ref, bn_ref, sel_ref, seg_ref,
             wx_ref, wa_ref, xreg_ref, j_ref, st_ref):
    w = w_ref[...]
    bn = bn_ref[...]
    wx = wx_ref[...]
    wa = wa_ref[...]
    ssum = None
    ssq = None
    sel = sel_ref[...]
    for g in range(x_ref.shape[0]):
        v = x_ref[g]
        m = jnp.maximum(v, _sh(v, 1))
        mb = m.astype(jnp.bfloat16)
        vb = v.astype(jnp.bfloat16)
        mb = jnp.maximum(mb, _sh(mb, 2))
        mb = jnp.maximum(mb, _sh(mb, 4))
        mb = jnp.maximum(jnp.maximum(mb, _sh(mb, 8)),
                         jnp.maximum(_sh(mb, 16), _sh(vb, 24)))
        xmax = jnp.dot(sel, mb, preferred_element_type=jnp.float32)
        xreg_ref[g] = jnp.dot(seg_ref[...], v,
                              preferred_element_type=jnp.float32)

        ap = jnp.max(a_ref[g], axis=1, keepdims=True)
        ap = jnp.log(ap + 1.0) * _LN10_INV
        y = (_shift_down(ap * w[0]) + ap * w[1] + _shift_up(ap * w[2]))
        a = jnp.maximum(y * bn[0:1, :] + bn[1:2, :], 0.0)

        y0 = (jnp.dot(xmax, wx[0], preferred_element_type=jnp.float32) +
              jnp.dot(a, wa[0], preferred_element_type=jnp.float32))
        y1 = (jnp.dot(xmax, wx[1], preferred_element_type=jnp.float32) +
              jnp.dot(a, wa[1], preferred_element_type=jnp.float32))
        y2 = (jnp.dot(xmax, wx[2], preferred_element_type=jnp.float32) +
              jnp.dot(a, wa[2], preferred_element_type=jnp.float32))
        j = _shift_down(y0) + y1 + _shift_up(y2)
        j_ref[g] = j
        js = jnp.sum(j, axis=0, keepdims=True)
        jq = jnp.sum(j * j, axis=0, keepdims=True)
        ssum = js if ssum is None else ssum + js
        ssq = jq if ssq is None else ssq + jq
    st_ref[0] = jnp.concatenate([ssum, ssq], axis=0)


def _k2_body(j_ref, st_ref, xreg_ref, o_ref):
    g_blk, t, _ = j_ref.shape
    n = st_ref.shape[0] * g_blk * t
    s = jnp.sum(st_ref[...], axis=0) * (1.0 / n)
    mean = s[0:1, :]
    var = jnp.maximum(s[1:2, :] - mean * mean, 0.0)
    sc = jax.lax.rsqrt(var + _EPS)
    bias = -mean * sc
    d = xreg_ref.shape[2]
    msk = _seg_mean_matrix(t)
    for g in range(g_blk):
        jn = jnp.maximum(j_ref[g] * sc + bias, 0.0)
        jr = jnp.dot(msk, jn, preferred_element_type=jnp.float32)
        jl = jnp.log(jr + 1.0) * _LN2_INV
        o_ref[g, :, :d] = xreg_ref[g]
        o_ref[g, :, d:] = jl


def kernel(x, atac, atac_conv_w3, joint_conv_w3_x, joint_conv_w3_a):
    B, L, D = x.shape
    T = L // _PATCH
    A = atac_conv_w3.shape[2]
    J = joint_conv_w3_x.shape[2]
    a3 = atac.reshape(B, T, _PATCH)

    tt = np.arange(L)
    sel = (tt[None, :] == (np.arange(T) * _PATCH)[:, None])
    seg = ((tt[None, :] // _SEG) == np.arange(_N_PEAKS)[:, None]).astype(
        np.float32) * (1.0 / _SEG)
    sel = jnp.asarray(sel, dtype=jnp.bfloat16)
    seg = jnp.asarray(seg)

    bn1 = pl.pallas_call(
        _k0_body,
        out_shape=jax.ShapeDtypeStruct((2, A), jnp.float32),
        grid=(1,),
        in_specs=[pl.BlockSpec((B, T, _PATCH), lambda i: (0, 0, 0)),
                  pl.BlockSpec((3, 1, A), lambda i: (0, 0, 0))],
        out_specs=pl.BlockSpec((2, A), lambda i: (0, 0)),
    )(a3, atac_conv_w3)

    G = next(g for g in (8, 4, 2, 1) if B % g == 0)
    NG = B // G
    xreg, j, st2 = pl.pallas_call(
        _k1_body,
        out_shape=(jax.ShapeDtypeStruct((B, _N_PEAKS, D), jnp.float32),
                   jax.ShapeDtypeStruct((B, T, J), jnp.float32),
                   jax.ShapeDtypeStruct((NG, 2, J), jnp.float32)),
        grid=(NG,),
        in_specs=[pl.BlockSpec((G, L, D), lambda b: (b, 0, 0)),
                  pl.BlockSpec((G, T, _PATCH), lambda b: (b, 0, 0)),
                  pl.BlockSpec((3, 1, A), lambda b: (0, 0, 0)),
                  pl.BlockSpec((2, A), lambda b: (0, 0)),
                  pl.BlockSpec((T, L), lambda b: (0, 0)),
                  pl.BlockSpec((_N_PEAKS, L), lambda b: (0, 0)),
                  pl.BlockSpec((3, D, J), lambda b: (0, 0, 0)),
                  pl.BlockSpec((3, A, J), lambda b: (0, 0, 0))],
        out_specs=[pl.BlockSpec((G, _N_PEAKS, D), lambda b: (b, 0, 0)),
                   pl.BlockSpec((G, T, J), lambda b: (b, 0, 0)),
                   pl.BlockSpec((1, 2, J), lambda b: (b, 0, 0))],
        compiler_params=pltpu.CompilerParams(
            dimension_semantics=("parallel",),
            vmem_limit_bytes=100 * 1024 * 1024),
    )(x, a3, atac_conv_w3, bn1, sel, seg, joint_conv_w3_x, joint_conv_w3_a)

    out = pl.pallas_call(
        _k2_body,
        out_shape=jax.ShapeDtypeStruct((B, _N_PEAKS, D + J), jnp.float32),
        grid=(NG,),
        in_specs=[pl.BlockSpec((G, T, J), lambda b: (b, 0, 0)),
                  pl.BlockSpec((NG, 2, J), lambda b: (0, 0, 0)),
                  pl.BlockSpec((G, _N_PEAKS, D), lambda b: (b, 0, 0))],
        out_specs=pl.BlockSpec((G, _N_PEAKS, D + J), lambda b: (b, 0, 0)),
        compiler_params=pltpu.CompilerParams(
            dimension_semantics=("parallel",)),
    )(j, st2, xreg)
    return out


# speedup vs baseline: 5.5044x; 1.0269x over previous
import math

import numpy as np

import jax
import jax.numpy as jnp
from jax.experimental import pallas as pl
from jax.experimental.pallas import tpu as pltpu

_PATCH = 25
_SEG = 400
_SEG_PATCHES = 16
_N_PEAKS = 4
_LN10_INV = 1.0 / math.log(10.0)
_LN2_INV = 1.0 / math.log(2.0)
_EPS = 1e-5


def _sh(y, s):
    return jnp.concatenate([y[s:], jnp.zeros((s, y.shape[1]), y.dtype)], axis=0)


def _shift_down(y):
    z = jnp.zeros_like(y[:1])
    return jnp.concatenate([z, y[:-1]], axis=0)


def _shift_up(y):
    z = jnp.zeros_like(y[:1])
    return jnp.concatenate([y[1:], z], axis=0)


def _seg_mean_matrix(t):
    r = jax.lax.broadcasted_iota(jnp.int32, (_N_PEAKS, t), 0)
    c = jax.lax.broadcasted_iota(jnp.int32, (_N_PEAKS, t), 1)
    msk = (c >= r * _SEG_PATCHES) & (c < (r + 1) * _SEG_PATCHES)
    return msk.astype(jnp.float32) * (1.0 / _SEG_PATCHES)


def _k0_body(a_ref, w_ref, bn_ref):
    af = jnp.max(a_ref[...], axis=2)
    af = jnp.log(af + 1.0) * _LN10_INV
    z = jnp.zeros_like(af[:, :1])
    u0 = jnp.concatenate([z, af[:, :-1]], axis=1)
    u2 = jnp.concatenate([af[:, 1:], z], axis=1)
    n = af.shape[0] * af.shape[1]
    inv_n = 1.0 / n
    s0 = jnp.sum(u0) * inv_n
    s1 = jnp.sum(af) * inv_n
    s2 = jnp.sum(u2) * inv_n
    q00 = jnp.sum(u0 * u0) * inv_n
    q11 = jnp.sum(af * af) * inv_n
    q22 = jnp.sum(u2 * u2) * inv_n
    q01 = jnp.sum(u0 * af) * inv_n
    q02 = jnp.sum(u0 * u2) * inv_n
    q12 = jnp.sum(af * u2) * inv_n
    w = w_ref[...]
    w0, w1, w2 = w[0], w[1], w[2]
    mean = w0 * s0 + w1 * s1 + w2 * s2
    ey2 = (w0 * w0 * q00 + w1 * w1 * q11 + w2 * w2 * q22 +
           2.0 * (w0 * w1 * q01 + w0 * w2 * q02 + w1 * w2 * q12))
    var = jnp.maximum(ey2 - mean * mean, 0.0)
    sc = jax.lax.rsqrt(var + _EPS)
    bn_ref[...] = jnp.concatenate([sc, -mean * sc], axis=0)


def _k1_body(x_ref, a_ref, w_ref, bn_ref, sel_ref, seg_ref,
             wx_ref, wa_ref, xreg_ref, j_ref, st_ref):
    w = w_ref[...]
    bn = bn_ref[...]
    wx = wx_ref[...]
    wa = wa_ref[...]
    ssum = None
    ssq = None
    sel = sel_ref[...]
    for g in range(x_ref.shape[0]):
        v = x_ref[g]
        vb = v.astype(jnp.bfloat16)
        mb = jnp.maximum(vb, _sh(vb, 1))
        mb = jnp.maximum(mb, _sh(mb, 2))
        mb = jnp.maximum(mb, _sh(mb, 4))
        mb = jnp.maximum(jnp.maximum(mb, _sh(mb, 8)),
                         jnp.maximum(_sh(mb, 16), _sh(vb, 24)))
        xmax = jnp.dot(sel, mb, preferred_element_type=jnp.float32)
        xreg_ref[g] = jnp.dot(seg_ref[...], v,
                              preferred_element_type=jnp.float32)

        ap = jnp.max(a_ref[g], axis=1, keepdims=True)
        ap = jnp.log(ap + 1.0) * _LN10_INV
        y = (_shift_down(ap * w[0]) + ap * w[1] + _shift_up(ap * w[2]))
        a = jnp.maximum(y * bn[0:1, :] + bn[1:2, :], 0.0)

        y0 = (jnp.dot(xmax, wx[0], preferred_element_type=jnp.float32) +
              jnp.dot(a, wa[0], preferred_element_type=jnp.float32))
        y1 = (jnp.dot(xmax, wx[1], preferred_element_type=jnp.float32) +
              jnp.dot(a, wa[1], preferred_element_type=jnp.float32))
        y2 = (jnp.dot(xmax, wx[2], preferred_element_type=jnp.float32) +
              jnp.dot(a, wa[2], preferred_element_type=jnp.float32))
        j = _shift_down(y0) + y1 + _shift_up(y2)
        j_ref[g] = j
        js = jnp.sum(j, axis=0, keepdims=True)
        jq = jnp.sum(j * j, axis=0, keepdims=True)
        ssum = js if ssum is None else ssum + js
        ssq = jq if ssq is None else ssq + jq
    st_ref[0] = jnp.concatenate([ssum, ssq], axis=0)


def _k2_body(j_ref, st_ref, xreg_ref, o_ref):
    g_blk, t, _ = j_ref.shape
    n = st_ref.shape[0] * g_blk * t
    s = jnp.sum(st_ref[...], axis=0) * (1.0 / n)
    mean = s[0:1, :]
    var = jnp.maximum(s[1:2, :] - mean * mean, 0.0)
    sc = jax.lax.rsqrt(var + _EPS)
    bias = -mean * sc
    d = xreg_ref.shape[2]
    msk = _seg_mean_matrix(t)
    for g in range(g_blk):
        jn = jnp.maximum(j_ref[g] * sc + bias, 0.0)
        jr = jnp.dot(msk, jn, preferred_element_type=jnp.float32)
        jl = jnp.log(jr + 1.0) * _LN2_INV
        o_ref[g, :, :d] = xreg_ref[g]
        o_ref[g, :, d:] = jl


def kernel(x, atac, atac_conv_w3, joint_conv_w3_x, joint_conv_w3_a):
    B, L, D = x.shape
    T = L // _PATCH
    A = atac_conv_w3.shape[2]
    J = joint_conv_w3_x.shape[2]
    a3 = atac.reshape(B, T, _PATCH)

    tt = np.arange(L)
    sel = (tt[None, :] == (np.arange(T) * _PATCH)[:, None])
    seg = ((tt[None, :] // _SEG) == np.arange(_N_PEAKS)[:, None]).astype(
        np.float32) * (1.0 / _SEG)
    sel = jnp.asarray(sel, dtype=jnp.bfloat16)
    seg = jnp.asarray(seg)

    bn1 = pl.pallas_call(
        _k0_body,
        out_shape=jax.ShapeDtypeStruct((2, A), jnp.float32),
        grid=(1,),
        in_specs=[pl.BlockSpec((B, T, _PATCH), lambda i: (0, 0, 0)),
                  pl.BlockSpec((3, 1, A), lambda i: (0, 0, 0))],
        out_specs=pl.BlockSpec((2, A), lambda i: (0, 0)),
    )(a3, atac_conv_w3)

    G = next(g for g in (8, 4, 2, 1) if B % g == 0)
    NG = B // G
    xreg, j, st2 = pl.pallas_call(
        _k1_body,
        out_shape=(jax.ShapeDtypeStruct((B, _N_PEAKS, D), jnp.float32),
                   jax.ShapeDtypeStruct((B, T, J), jnp.float32),
                   jax.ShapeDtypeStruct((NG, 2, J), jnp.float32)),
        grid=(NG,),
        in_specs=[pl.BlockSpec((G, L, D), lambda b: (b, 0, 0)),
                  pl.BlockSpec((G, T, _PATCH), lambda b: (b, 0, 0)),
                  pl.BlockSpec((3, 1, A), lambda b: (0, 0, 0)),
                  pl.BlockSpec((2, A), lambda b: (0, 0)),
                  pl.BlockSpec((T, L), lambda b: (0, 0)),
                  pl.BlockSpec((_N_PEAKS, L), lambda b: (0, 0)),
                  pl.BlockSpec((3, D, J), lambda b: (0, 0, 0)),
                  pl.BlockSpec((3, A, J), lambda b: (0, 0, 0))],
        out_specs=[pl.BlockSpec((G, _N_PEAKS, D), lambda b: (b, 0, 0)),
                   pl.BlockSpec((G, T, J), lambda b: (b, 0, 0)),
                   pl.BlockSpec((1, 2, J), lambda b: (b, 0, 0))],
        compiler_params=pltpu.CompilerParams(
            dimension_semantics=("parallel",),
            vmem_limit_bytes=100 * 1024 * 1024),
    )(x, a3, atac_conv_w3, bn1, sel, seg, joint_conv_w3_x, joint_conv_w3_a)

    out = pl.pallas_call(
        _k2_body,
        out_shape=jax.ShapeDtypeStruct((B, _N_PEAKS, D + J), jnp.float32),
        grid=(NG,),
        in_specs=[pl.BlockSpec((G, T, J), lambda b: (b, 0, 0)),
                  pl.BlockSpec((NG, 2, J), lambda b: (0, 0, 0)),
                  pl.BlockSpec((G, _N_PEAKS, D), lambda b: (b, 0, 0))],
        out_specs=pl.BlockSpec((G, _N_PEAKS, D + J), lambda b: (b, 0, 0)),
        compiler_params=pltpu.CompilerParams(
            dimension_semantics=("parallel",)),
    )(j, st2, xreg)
    return out


# speedup vs baseline: 5.5334x; 1.0053x over previous
import math

import numpy as np

import jax
import jax.numpy as jnp
from jax.experimental import pallas as pl
from jax.experimental.pallas import tpu as pltpu

_PATCH = 25
_SEG = 400
_SEG_PATCHES = 16
_N_PEAKS = 4
_LN10_INV = 1.0 / math.log(10.0)
_LN2_INV = 1.0 / math.log(2.0)
_EPS = 1e-5


def _sh(y, s):
    return jnp.concatenate([y[s:], jnp.zeros((s, y.shape[1]), y.dtype)], axis=0)


def _shift_down(y):
    z = jnp.zeros_like(y[:1])
    return jnp.concatenate([z, y[:-1]], axis=0)


def _shift_up(y):
    z = jnp.zeros_like(y[:1])
    return jnp.concatenate([y[1:], z], axis=0)


def _seg_mean_matrix(t):
    r = jax.lax.broadcasted_iota(jnp.int32, (_N_PEAKS, t), 0)
    c = jax.lax.broadcasted_iota(jnp.int32, (_N_PEAKS, t), 1)
    msk = (c >= r * _SEG_PATCHES) & (c < (r + 1) * _SEG_PATCHES)
    return msk.astype(jnp.float32) * (1.0 / _SEG_PATCHES)


def _k0_body(a_ref, w_ref, bn_ref):
    af = jnp.max(a_ref[...], axis=2)
    af = jnp.log(af + 1.0) * _LN10_INV
    z = jnp.zeros_like(af[:, :1])
    u0 = jnp.concatenate([z, af[:, :-1]], axis=1)
    u2 = jnp.concatenate([af[:, 1:], z], axis=1)
    n = af.shape[0] * af.shape[1]
    inv_n = 1.0 / n
    s0 = jnp.sum(u0) * inv_n
    s1 = jnp.sum(af) * inv_n
    s2 = jnp.sum(u2) * inv_n
    q00 = jnp.sum(u0 * u0) * inv_n
    q11 = jnp.sum(af * af) * inv_n
    q22 = jnp.sum(u2 * u2) * inv_n
    q01 = jnp.sum(u0 * af) * inv_n
    q02 = jnp.sum(u0 * u2) * inv_n
    q12 = jnp.sum(af * u2) * inv_n
    w = w_ref[...]
    w0, w1, w2 = w[0], w[1], w[2]
    mean = w0 * s0 + w1 * s1 + w2 * s2
    ey2 = (w0 * w0 * q00 + w1 * w1 * q11 + w2 * w2 * q22 +
           2.0 * (w0 * w1 * q01 + w0 * w2 * q02 + w1 * w2 * q12))
    var = jnp.maximum(ey2 - mean * mean, 0.0)
    sc = jax.lax.rsqrt(var + _EPS)
    bn_ref[...] = jnp.concatenate([sc, -mean * sc], axis=0)


def _k1_body(x_ref, a_ref, w_ref, bn_ref, sel_ref, seg_ref,
             wx_ref, wa_ref, xreg_ref, j_ref, st_ref):
    w = w_ref[...]
    bn = bn_ref[...]
    wx = wx_ref[...]
    wa = wa_ref[...]
    ssum = None
    ssq = None
    sel = sel_ref[...]
    for g in range(x_ref.shape[0]):
        v = x_ref[g]
        vb = v.astype(jnp.bfloat16)
        mb = jnp.maximum(vb, _sh(vb, 1))
        mb = jnp.maximum(mb, _sh(mb, 2))
        mb = jnp.maximum(mb, _sh(mb, 4))
        mb = jnp.maximum(jnp.maximum(mb, _sh(mb, 8)),
                         jnp.maximum(_sh(mb, 16), _sh(vb, 24)))
        xmax = jnp.dot(sel, mb, preferred_element_type=jnp.float32)
        xreg_ref[g] = jnp.dot(seg_ref[...], v,
                              preferred_element_type=jnp.float32)

        ap = jnp.max(a_ref[g], axis=1, keepdims=True)
        ap = jnp.log(ap + 1.0) * _LN10_INV
        y = (_shift_down(ap * w[0]) + ap * w[1] + _shift_up(ap * w[2]))
        a = jnp.maximum(y * bn[0:1, :] + bn[1:2, :], 0.0)

        y0 = (jnp.dot(xmax, wx[0], preferred_element_type=jnp.float32) +
              jnp.dot(a, wa[0], preferred_element_type=jnp.float32))
        y1 = (jnp.dot(xmax, wx[1], preferred_element_type=jnp.float32) +
              jnp.dot(a, wa[1], preferred_element_type=jnp.float32))
        y2 = (jnp.dot(xmax, wx[2], preferred_element_type=jnp.float32) +
              jnp.dot(a, wa[2], preferred_element_type=jnp.float32))
        j = _shift_down(y0) + y1 + _shift_up(y2)
        j_ref[g] = j
        js = jnp.sum(j, axis=0, keepdims=True)
        jq = jnp.sum(j * j, axis=0, keepdims=True)
        ssum = js if ssum is None else ssum + js
        ssq = jq if ssq is None else ssq + jq
    st_ref[0] = jnp.concatenate([ssum, ssq], axis=0)


def _k2_body(j_ref, st_ref, xreg_ref, o_ref):
    g_blk, t, _ = j_ref.shape
    n = st_ref.shape[0] * g_blk * t
    s = jnp.sum(st_ref[...], axis=0) * (1.0 / n)
    mean = s[0:1, :]
    var = jnp.maximum(s[1:2, :] - mean * mean, 0.0)
    sc = jax.lax.rsqrt(var + _EPS)
    bias = -mean * sc
    d = xreg_ref.shape[2]
    msk = _seg_mean_matrix(t)
    for g in range(g_blk):
        jn = jnp.maximum(j_ref[g] * sc + bias, 0.0)
        jr = jnp.dot(msk, jn, preferred_element_type=jnp.float32)
        jl = jnp.log(jr + 1.0) * _LN2_INV
        o_ref[g, :, :d] = xreg_ref[g]
        o_ref[g, :, d:] = jl


def kernel(x, atac, atac_conv_w3, joint_conv_w3_x, joint_conv_w3_a):
    B, L, D = x.shape
    T = L // _PATCH
    A = atac_conv_w3.shape[2]
    J = joint_conv_w3_x.shape[2]
    a3 = atac.reshape(B, T, _PATCH)

    tt = np.arange(L)
    sel = (tt[None, :] == (np.arange(T) * _PATCH)[:, None])
    seg = ((tt[None, :] // _SEG) == np.arange(_N_PEAKS)[:, None]).astype(
        np.float32) * (1.0 / _SEG)
    sel = jnp.asarray(sel, dtype=jnp.bfloat16)
    seg = jnp.asarray(seg)

    bn1 = pl.pallas_call(
        _k0_body,
        out_shape=jax.ShapeDtypeStruct((2, A), jnp.float32),
        grid=(1,),
        in_specs=[pl.BlockSpec((B, T, _PATCH), lambda i: (0, 0, 0)),
                  pl.BlockSpec((3, 1, A), lambda i: (0, 0, 0))],
        out_specs=pl.BlockSpec((2, A), lambda i: (0, 0)),
    )(a3, atac_conv_w3)

    G = next(g for g in (16, 8, 4, 2, 1) if B % g == 0)
    NG = B // G
    xreg, j, st2 = pl.pallas_call(
        _k1_body,
        out_shape=(jax.ShapeDtypeStruct((B, _N_PEAKS, D), jnp.float32),
                   jax.ShapeDtypeStruct((B, T, J), jnp.float32),
                   jax.ShapeDtypeStruct((NG, 2, J), jnp.float32)),
        grid=(NG,),
        in_specs=[pl.BlockSpec((G, L, D), lambda b: (b, 0, 0)),
                  pl.BlockSpec((G, T, _PATCH), lambda b: (b, 0, 0)),
                  pl.BlockSpec((3, 1, A), lambda b: (0, 0, 0)),
                  pl.BlockSpec((2, A), lambda b: (0, 0)),
                  pl.BlockSpec((T, L), lambda b: (0, 0)),
                  pl.BlockSpec((_N_PEAKS, L), lambda b: (0, 0)),
                  pl.BlockSpec((3, D, J), lambda b: (0, 0, 0)),
                  pl.BlockSpec((3, A, J), lambda b: (0, 0, 0))],
        out_specs=[pl.BlockSpec((G, _N_PEAKS, D), lambda b: (b, 0, 0)),
                   pl.BlockSpec((G, T, J), lambda b: (b, 0, 0)),
                   pl.BlockSpec((1, 2, J), lambda b: (b, 0, 0))],
        compiler_params=pltpu.CompilerParams(
            dimension_semantics=("parallel",),
            vmem_limit_bytes=100 * 1024 * 1024),
    )(x, a3, atac_conv_w3, bn1, sel, seg, joint_conv_w3_x, joint_conv_w3_a)

    out = pl.pallas_call(
        _k2_body,
        out_shape=jax.ShapeDtypeStruct((B, _N_PEAKS, D + J), jnp.float32),
        grid=(NG,),
        in_specs=[pl.BlockSpec((G, T, J), lambda b: (b, 0, 0)),
                  pl.BlockSpec((NG, 2, J), lambda b: (0, 0, 0)),
                  pl.BlockSpec((G, _N_PEAKS, D), lambda b: (b, 0, 0))],
        out_specs=pl.BlockSpec((G, _N_PEAKS, D + J), lambda b: (b, 0, 0)),
        compiler_params=pltpu.CompilerParams(
            dimension_semantics=("parallel",)),
    )(j, st2, xreg)
    return out
